# baseline (speedup 1.0000x reference)
import jax
import jax.numpy as jnp
from jax.experimental import pallas as pl
from jax.experimental.pallas import tpu as pltpu

B, S, D = 1, 2048, 2048
E, K = 8, 2
M = 4096
G = 4
GS = (B * S) // G
CAP = int(GS * K / E * 1.25)


def _route(xg, wr):
    logits = jax.lax.dot_general(xg, wr, (((1,), (0,)), ((), ())),
                                 preferred_element_type=jnp.float32)
    lmax = jnp.max(logits, axis=1, keepdims=True)
    ex = jnp.exp(logits - lmax)
    probs = ex / jnp.sum(ex, axis=1, keepdims=True)

    iota_e = jax.lax.broadcasted_iota(jnp.int32, (GS, E), 1)
    m1 = jnp.max(probs, axis=1, keepdims=True)
    a1 = jnp.min(jnp.where(probs == m1, iota_e, E), axis=1, keepdims=True)
    probs2 = jnp.where(iota_e == a1, -jnp.inf, probs)
    m2 = jnp.max(probs2, axis=1, keepdims=True)
    a2 = jnp.min(jnp.where(probs2 == m2, iota_e, E), axis=1, keepdims=True)

    oh0 = (iota_e == a1).astype(jnp.float32)
    oh1 = (iota_e == a2).astype(jnp.float32)
    ri = jax.lax.broadcasted_iota(jnp.int32, (GS, GS), 0)
    ci = jax.lax.broadcasted_iota(jnp.int32, (GS, GS), 1)
    tri = (ci <= ri).astype(jnp.float32)
    c0 = jax.lax.dot_general(tri, oh0, (((1,), (0,)), ((), ())),
                             preferred_element_type=jnp.float32)
    c1 = jax.lax.dot_general(tri, oh1, (((1,), (0,)), ((), ())),
                             preferred_element_type=jnp.float32)
    total0 = c0[GS - 1:GS, :]
    pos0 = jnp.sum(c0 * oh0, axis=1, keepdims=True) - 1.0
    pos1 = jnp.sum((c1 + total0) * oh1, axis=1, keepdims=True) - 1.0
    return a1, a2, m1, m2, pos0, pos1


def _router_body(xg_ref, wr_ref, xe_ref, comb_ref):
    e = pl.program_id(1)
    xg = xg_ref[0]
    a1, a2, m1, m2, pos0, pos1 = _route(xg, wr_ref[...])

    iota_c = jax.lax.broadcasted_iota(jnp.int32, (GS, CAP), 1)
    hit0 = (a1 == e) & (iota_c == pos0.astype(jnp.int32)) & (pos0 < CAP)
    hit1 = (a2 == e) & (iota_c == pos1.astype(jnp.int32)) & (pos1 < CAP)
    disp = hit0.astype(jnp.float32) + hit1.astype(jnp.float32)
    comb = jnp.where(hit0, m1, 0.0) + jnp.where(hit1, m2, 0.0)
    comb_ref[0, 0, 0] = comb.astype(jnp.bfloat16)
    xe = jax.lax.dot_general(disp, xg, (((0,), (0,)), ((), ())),
                             preferred_element_type=jnp.float32)
    xe_ref[0] = xe.astype(jnp.bfloat16)


BM = 256


def _ffn_body(xe_ref, w0_ref, w1_ref, wo_ref, comb_ref, y_ref, oe_ref):
    e = pl.program_id(0)
    mt = pl.program_id(1)
    a = xe_ref[0]
    h0 = jax.lax.dot_general(a, w0_ref[0], (((1,), (0,)), ((), ())),
                             preferred_element_type=jnp.float32)
    h1 = jax.lax.dot_general(a, w1_ref[0], (((1,), (0,)), ((), ())),
                             preferred_element_type=jnp.float32)
    h = (h0 * jax.lax.logistic(h0)) * h1
    out = jax.lax.dot_general(h, wo_ref[0], (((1,), (0,)), ((), ())),
                              preferred_element_type=jnp.float32)

    @pl.when(mt == 0)
    def _():
        oe_ref[...] = out

    @pl.when(mt > 0)
    def _():
        oe_ref[...] += out

    @pl.when(mt == M // BM - 1)
    def _():
        oe = oe_ref[...].astype(jnp.bfloat16)
        for g in range(G):
            yg = jax.lax.dot_general(
                comb_ref[0, 0, g], oe[g * CAP:(g + 1) * CAP, :],
                (((1,), (0,)), ((), ())),
                preferred_element_type=jnp.float32)

            @pl.when(e == 0)
            def _():
                y_ref[g] = yg

            @pl.when(e > 0)
            def _():
                y_ref[g] += yg


@jax.jit
def kernel(x, w_router, wi_0, wi_1, wo):
    xg = x.reshape(G, GS, D)

    xe, comb = pl.pallas_call(
        _router_body,
        grid=(G, E),
        in_specs=[
            pl.BlockSpec((1, GS, D), lambda g, e: (g, 0, 0)),
            pl.BlockSpec((D, E), lambda g, e: (0, 0)),
        ],
        out_specs=[
            pl.BlockSpec((1, CAP, D), lambda g, e: (e, g, 0)),
            pl.BlockSpec((1, 1, 1, GS, CAP), lambda g, e: (e, 0, g, 0, 0)),
        ],
        out_shape=[
            jax.ShapeDtypeStruct((E, G * CAP, D), jnp.bfloat16),
            jax.ShapeDtypeStruct((E, 1, G, GS, CAP), jnp.bfloat16),
        ],
    )(xg, w_router)

    y = pl.pallas_call(
        _ffn_body,
        grid=(E, M // BM),
        in_specs=[
            pl.BlockSpec((1, G * CAP, D), lambda e, mt: (e, 0, 0)),
            pl.BlockSpec((1, D, BM), lambda e, mt: (e, 0, mt)),
            pl.BlockSpec((1, D, BM), lambda e, mt: (e, 0, mt)),
            pl.BlockSpec((1, BM, D), lambda e, mt: (e, mt, 0)),
            pl.BlockSpec((1, 1, G, GS, CAP), lambda e, mt: (e, 0, 0, 0, 0)),
        ],
        out_specs=pl.BlockSpec((G, GS, D), lambda e, mt: (0, 0, 0)),
        out_shape=jax.ShapeDtypeStruct((G, GS, D), jnp.float32),
        scratch_shapes=[pltpu.VMEM((G * CAP, D), jnp.float32)],
    )(xe, wi_0, wi_1, wo, comb)

    return y.reshape(B, S, D)


# speedup vs baseline: 1.2856x; 1.2856x over previous
import jax
import jax.numpy as jnp
from jax.experimental import pallas as pl
from jax.experimental.pallas import tpu as pltpu

B, S, D = 1, 2048, 2048
E, K = 8, 2
M = 4096
G = 4
GS = (B * S) // G
CAP = int(GS * K / E * 1.25)


def _router_body(xg_ref, wr_ref, xe_ref, comb_ref):
    xg = xg_ref[0]
    logits = jax.lax.dot_general(xg, wr_ref[...], (((1,), (0,)), ((), ())),
                                 preferred_element_type=jnp.float32)
    lmax = jnp.max(logits, axis=1, keepdims=True)
    ex = jnp.exp(logits - lmax)
    probs = ex / jnp.sum(ex, axis=1, keepdims=True)

    iota_e = jax.lax.broadcasted_iota(jnp.int32, (GS, E), 1)
    m1 = jnp.max(probs, axis=1, keepdims=True)
    a1 = jnp.min(jnp.where(probs == m1, iota_e, E), axis=1, keepdims=True)
    probs2 = jnp.where(iota_e == a1, -jnp.inf, probs)
    m2 = jnp.max(probs2, axis=1, keepdims=True)
    a2 = jnp.min(jnp.where(probs2 == m2, iota_e, E), axis=1, keepdims=True)

    oh0 = (iota_e == a1).astype(jnp.float32)
    oh1 = (iota_e == a2).astype(jnp.float32)
    ri = jax.lax.broadcasted_iota(jnp.int32, (GS, GS), 0)
    ci = jax.lax.broadcasted_iota(jnp.int32, (GS, GS), 1)
    tri = (ci <= ri).astype(jnp.float32)
    c0 = jax.lax.dot_general(tri, oh0, (((1,), (0,)), ((), ())),
                             preferred_element_type=jnp.float32)
    c1 = jax.lax.dot_general(tri, oh1, (((1,), (0,)), ((), ())),
                             preferred_element_type=jnp.float32)
    total0 = c0[GS - 1:GS, :]
    pos0i = (jnp.sum(c0 * oh0, axis=1, keepdims=True) - 1.0).astype(jnp.int32)
    pos1i = (jnp.sum((c1 + total0) * oh1, axis=1, keepdims=True)
             - 1.0).astype(jnp.int32)

    iota_c = jax.lax.broadcasted_iota(jnp.int32, (GS, CAP), 1)
    keep0 = pos0i < CAP
    keep1 = pos1i < CAP
    for e in range(E):
        hit0 = (a1 == e) & (iota_c == pos0i) & keep0
        hit1 = (a2 == e) & (iota_c == pos1i) & keep1
        disp = hit0.astype(jnp.float32) + hit1.astype(jnp.float32)
        comb = jnp.where(hit0, m1, 0.0) + jnp.where(hit1, m2, 0.0)
        comb_ref[0, e] = comb.astype(jnp.bfloat16)
        xe = jax.lax.dot_general(disp, xg, (((0,), (0,)), ((), ())),
                                 preferred_element_type=jnp.float32)
        xe_ref[0, e] = xe.astype(jnp.bfloat16)


BM = 512


def _ffn_body(xe_ref, w0_ref, w1_ref, wo_ref, oe_ref, acc_ref):
    mt = pl.program_id(1)
    a = xe_ref[...].reshape(G * CAP, D)
    h0 = jax.lax.dot_general(a, w0_ref[0], (((1,), (0,)), ((), ())),
                             preferred_element_type=jnp.float32)
    h1 = jax.lax.dot_general(a, w1_ref[0], (((1,), (0,)), ((), ())),
                             preferred_element_type=jnp.float32)
    h = (h0 * jax.lax.logistic(h0)) * h1
    out = jax.lax.dot_general(h, wo_ref[0], (((1,), (0,)), ((), ())),
                              preferred_element_type=jnp.float32)

    @pl.when(mt == 0)
    def _():
        acc_ref[...] = out

    @pl.when(mt > 0)
    def _():
        acc_ref[...] += out

    @pl.when(mt == M // BM - 1)
    def _():
        oe_ref[0] = acc_ref[...].astype(jnp.bfloat16)


def _combine_body(comb_ref, oe_ref, y_ref):
    e = pl.program_id(1)
    y = jax.lax.dot_general(comb_ref[0, 0], oe_ref[0],
                            (((1,), (0,)), ((), ())),
                            preferred_element_type=jnp.float32)

    @pl.when(e == 0)
    def _():
        y_ref[0] = y

    @pl.when(e > 0)
    def _():
        y_ref[0] += y


@jax.jit
def kernel(x, w_router, wi_0, wi_1, wo):
    xg = x.reshape(G, GS, D)

    xe, comb = pl.pallas_call(
        _router_body,
        grid=(G,),
        in_specs=[
            pl.BlockSpec((1, GS, D), lambda g: (g, 0, 0)),
            pl.BlockSpec((D, E), lambda g: (0, 0)),
        ],
        out_specs=[
            pl.BlockSpec((1, E, CAP, D), lambda g: (g, 0, 0, 0)),
            pl.BlockSpec((1, E, GS, CAP), lambda g: (g, 0, 0, 0)),
        ],
        out_shape=[
            jax.ShapeDtypeStruct((G, E, CAP, D), jnp.bfloat16),
            jax.ShapeDtypeStruct((G, E, GS, CAP), jnp.bfloat16),
        ],
    )(xg, w_router)

    oe = pl.pallas_call(
        _ffn_body,
        grid=(E, M // BM),
        in_specs=[
            pl.BlockSpec((G, 1, CAP, D), lambda e, mt: (0, e, 0, 0)),
            pl.BlockSpec((1, D, BM), lambda e, mt: (e, 0, mt)),
            pl.BlockSpec((1, D, BM), lambda e, mt: (e, 0, mt)),
            pl.BlockSpec((1, BM, D), lambda e, mt: (e, mt, 0)),
        ],
        out_specs=pl.BlockSpec((1, G * CAP, D), lambda e, mt: (e, 0, 0)),
        out_shape=jax.ShapeDtypeStruct((E, G * CAP, D), jnp.bfloat16),
        scratch_shapes=[pltpu.VMEM((G * CAP, D), jnp.float32)],
    )(xe, wi_0, wi_1, wo)

    y = pl.pallas_call(
        _combine_body,
        grid=(G, E),
        in_specs=[
            pl.BlockSpec((1, 1, GS, CAP), lambda g, e: (g, e, 0, 0)),
            pl.BlockSpec((1, CAP, D), lambda g, e: (e, g, 0)),
        ],
        out_specs=pl.BlockSpec((1, GS, D), lambda g, e: (g, 0, 0)),
        out_shape=jax.ShapeDtypeStruct((G, GS, D), jnp.float32),
    )(comb, oe)

    return y.reshape(B, S, D)


# speedup vs baseline: 1.3834x; 1.0761x over previous
import jax
import jax.numpy as jnp
from jax.experimental import pallas as pl
from jax.experimental.pallas import tpu as pltpu

B, S, D = 1, 2048, 2048
E, K = 8, 2
M = 4096
G = 4
GS = (B * S) // G
CAP = int(GS * K / E * 1.25)
EC = E * CAP


def _router_body(xg_ref, wr_ref, xe_ref, comb_ref):
    xg = xg_ref[0]
    logits = jax.lax.dot_general(xg, wr_ref[...], (((1,), (0,)), ((), ())),
                                 preferred_element_type=jnp.float32)
    lmax = jnp.max(logits, axis=1, keepdims=True)
    ex = jnp.exp(logits - lmax)
    probs = ex / jnp.sum(ex, axis=1, keepdims=True)

    iota_e = jax.lax.broadcasted_iota(jnp.int32, (GS, E), 1)
    m1 = jnp.max(probs, axis=1, keepdims=True)
    a1 = jnp.min(jnp.where(probs == m1, iota_e, E), axis=1, keepdims=True)
    probs2 = jnp.where(iota_e == a1, -jnp.inf, probs)
    m2 = jnp.max(probs2, axis=1, keepdims=True)
    a2 = jnp.min(jnp.where(probs2 == m2, iota_e, E), axis=1, keepdims=True)

    oh0 = (iota_e == a1).astype(jnp.float32)
    oh1 = (iota_e == a2).astype(jnp.float32)
    ri = jax.lax.broadcasted_iota(jnp.int32, (GS, GS), 0)
    ci = jax.lax.broadcasted_iota(jnp.int32, (GS, GS), 1)
    tri = (ci <= ri).astype(jnp.float32)
    c0 = jax.lax.dot_general(tri, oh0, (((1,), (0,)), ((), ())),
                             preferred_element_type=jnp.float32)
    c1 = jax.lax.dot_general(tri, oh1, (((1,), (0,)), ((), ())),
                             preferred_element_type=jnp.float32)
    total0 = c0[GS - 1:GS, :]
    pos0i = (jnp.sum(c0 * oh0, axis=1, keepdims=True) - 1.0).astype(jnp.int32)
    pos1i = (jnp.sum((c1 + total0) * oh1, axis=1, keepdims=True)
             - 1.0).astype(jnp.int32)

    j0 = jnp.where(pos0i < CAP, a1 * CAP + pos0i, -1)
    j1 = jnp.where(pos1i < CAP, a2 * CAP + pos1i, -1)

    iota_c = jax.lax.broadcasted_iota(jnp.int32, (GS, CAP), 1)
    for e in range(E):
        hit0 = (iota_c + e * CAP) == j0
        hit1 = (iota_c + e * CAP) == j1
        disp = hit0.astype(jnp.float32) + hit1.astype(jnp.float32)
        xe = jax.lax.dot_general(disp, xg, (((0,), (0,)), ((), ())),
                                 preferred_element_type=jnp.float32)
        xe_ref[e] = xe.astype(jnp.bfloat16)

    iota_j = jax.lax.broadcasted_iota(jnp.int32, (GS, EC), 1)
    comb = (jnp.where(iota_j == j0, m1, 0.0)
            + jnp.where(iota_j == j1, m2, 0.0))
    comb_ref[0] = comb.astype(jnp.bfloat16)


BM = 512


def _ffn_body(xe_ref, w0_ref, w1_ref, wo_ref, oe_ref, acc_ref):
    mt = pl.program_id(1)
    a = xe_ref[0]
    h0 = jax.lax.dot_general(a, w0_ref[0], (((1,), (0,)), ((), ())),
                             preferred_element_type=jnp.float32)
    h1 = jax.lax.dot_general(a, w1_ref[0], (((1,), (0,)), ((), ())),
                             preferred_element_type=jnp.float32)
    h = (h0 * jax.lax.logistic(h0)) * h1
    out = jax.lax.dot_general(h, wo_ref[0], (((1,), (0,)), ((), ())),
                              preferred_element_type=jnp.float32)

    @pl.when(mt == 0)
    def _():
        acc_ref[...] = out

    @pl.when(mt > 0)
    def _():
        acc_ref[...] += out

    @pl.when(mt == M // BM - 1)
    def _():
        oe_ref[...] = acc_ref[...].astype(jnp.bfloat16).reshape(G, 1, CAP, D)


def _combine_body(comb_ref, oe_ref, y_ref):
    oe = oe_ref[0].reshape(EC, D)
    y_ref[0] = jax.lax.dot_general(comb_ref[0], oe, (((1,), (0,)), ((), ())),
                                   preferred_element_type=jnp.float32)


@jax.jit
def kernel(x, w_router, wi_0, wi_1, wo):
    xg = x.reshape(G, GS, D)

    xe, comb = pl.pallas_call(
        _router_body,
        grid=(G,),
        in_specs=[
            pl.BlockSpec((1, GS, D), lambda g: (g, 0, 0)),
            pl.BlockSpec((D, E), lambda g: (0, 0)),
        ],
        out_specs=[
            pl.BlockSpec((E, CAP, D), lambda g: (0, g, 0)),
            pl.BlockSpec((1, GS, EC), lambda g: (g, 0, 0)),
        ],
        out_shape=[
            jax.ShapeDtypeStruct((E, G * CAP, D), jnp.bfloat16),
            jax.ShapeDtypeStruct((G, GS, EC), jnp.bfloat16),
        ],
    )(xg, w_router)

    oe = pl.pallas_call(
        _ffn_body,
        grid=(E, M // BM),
        in_specs=[
            pl.BlockSpec((1, G * CAP, D), lambda e, mt: (e, 0, 0)),
            pl.BlockSpec((1, D, BM), lambda e, mt: (e, 0, mt)),
            pl.BlockSpec((1, D, BM), lambda e, mt: (e, 0, mt)),
            pl.BlockSpec((1, BM, D), lambda e, mt: (e, mt, 0)),
        ],
        out_specs=pl.BlockSpec((G, 1, CAP, D), lambda e, mt: (0, e, 0, 0)),
        out_shape=jax.ShapeDtypeStruct((G, E, CAP, D), jnp.bfloat16),
        scratch_shapes=[pltpu.VMEM((G * CAP, D), jnp.float32)],
    )(xe, wi_0, wi_1, wo)

    y = pl.pallas_call(
        _combine_body,
        grid=(G,),
        in_specs=[
            pl.BlockSpec((1, GS, EC), lambda g: (g, 0, 0)),
            pl.BlockSpec((1, E, CAP, D), lambda g: (g, 0, 0, 0)),
        ],
        out_specs=pl.BlockSpec((1, GS, D), lambda g: (g, 0, 0)),
        out_shape=jax.ShapeDtypeStruct((G, GS, D), jnp.float32),
    )(comb, oe)

    return y.reshape(B, S, D)
